# baseline (speedup 1.0000x reference)
import functools
import math

import jax
import jax.numpy as jnp
from jax import lax
from jax.experimental import pallas as pl
from jax.experimental.pallas import tpu as pltpu
from jax.experimental.pallas import tpu_sc as plsc

_K = 16
_R = 128
_NW = 32
_CW = 80
_CH = 4
_PW = _CW * _CH
_CONV_T = 512
_HI = float("inf")


_L = 256
_D = 4
_BIG = 2**30


def _topk_body(n_valid, xi_ref, xt_ref, idx_ref):
    c = xi_ref.shape[1]
    n_pad = xt_ref.shape[0]
    G = n_pad // _L
    xi = xi_ref[...]
    xi = xi / (jnp.sqrt(jnp.sum(xi * xi, axis=1, keepdims=True)) + 1e-12)
    sqi = jnp.sum(xi * xi, axis=1, keepdims=True)
    laneiota = lax.broadcasted_iota(jnp.int32, (_R, _L), 1)
    kiota = lax.broadcasted_iota(jnp.int32, (_R, _K), 1)
    ones = jnp.ones((1, c), jnp.float32)

    def traversal(cutv, cutj):
        def gstep(g, bufs):
            bv = list(bufs[:_D])
            bj = list(bufs[_D:])
            xj = xt_ref[pl.ds(g * _L, _L), :]
            xj = xj / (jnp.sqrt(jnp.sum(xj * xj, axis=1, keepdims=True))
                       + 1e-12)
            inner = lax.dot_general(
                xi, xj, (((1,), (1,)), ((), ())),
                precision=lax.Precision.DEFAULT,
                preferred_element_type=jnp.float32)
            sqj = lax.dot_general(
                ones, xj * xj, (((1,), (1,)), ((), ())),
                precision=lax.Precision.HIGHEST,
                preferred_element_type=jnp.float32)
            d = sqi + (-2.0) * inner + sqj
            j = laneiota + g * _L
            exc = ((j >= n_valid) | (d < cutv)
                   | ((d == cutv) & (j <= cutj)))
            cv = jnp.where(exc, _HI, d)
            cj = j
            for i in range(_D):
                lt = (cv < bv[i]) | ((cv == bv[i]) & (cj < bj[i]))
                nv = jnp.minimum(cv, bv[i])
                nj = jnp.where(lt, cj, bj[i])
                cj = jnp.where(lt, bj[i], cj)
                cv = jnp.maximum(cv, bv[i])
                bv[i], bj[i] = nv, nj
            return tuple(bv) + tuple(bj)

        init = (tuple(jnp.full((_R, _L), _HI, jnp.float32)
                      for _ in range(_D))
                + tuple(jnp.full((_R, _L), _BIG, jnp.int32)
                        for _ in range(_D)))
        return lax.fori_loop(0, G, gstep, init)

    def extract(t, s):
        bv = list(s[:_D])
        bj = list(s[_D:2 * _D])
        cutv, cutj, cnt, kdone, acc = s[2 * _D:]
        hv, hj = bv[0], bj[0]
        m = jnp.min(hv, axis=1, keepdims=True)
        fl = jnp.min(jnp.where(hv == m, hj, _BIG), axis=1, keepdims=True)
        exh = cnt >= _D
        ev = jnp.min(jnp.where(exh, cutv, _HI), axis=1, keepdims=True)
        ej = jnp.min(jnp.where(exh & (cutv == ev), cutj, _BIG),
                     axis=1, keepdims=True)
        blocked = (m > ev) | ((m == ev) & (fl > ej))
        ok = (kdone < _K) & jnp.logical_not(blocked)
        acc = jnp.where((kiota == kdone) & ok, fl, acc)
        msk = ok & (hj == fl)
        oldv, oldj = bv[0], bj[0]
        for i in range(_D - 1):
            bv[i] = jnp.where(msk, bv[i + 1], bv[i])
            bj[i] = jnp.where(msk, bj[i + 1], bj[i])
        bv[_D - 1] = jnp.where(msk, _HI, bv[_D - 1])
        bj[_D - 1] = jnp.where(msk, _BIG, bj[_D - 1])
        cutv = jnp.where(msk, oldv, cutv)
        cutj = jnp.where(msk, oldj, cutj)
        cnt = cnt + msk.astype(jnp.int32)
        kdone = kdone + ok.astype(jnp.int32)
        return tuple(bv) + tuple(bj) + (cutv, cutj, cnt, kdone, acc)

    def phase(s):
        cutv, cutj, kdone, acc = s
        bufs = traversal(cutv, cutj)
        cnt = jnp.zeros((_R, _L), jnp.int32)
        out = lax.fori_loop(0, _K, extract,
                            bufs + (cutv, cutj, cnt, kdone, acc))
        return out[2 * _D], out[2 * _D + 1], out[2 * _D + 3], out[2 * _D + 4]

    def not_done(s):
        return jnp.min(s[2]) < _K

    final = lax.while_loop(
        not_done, phase,
        (jnp.full((_R, _L), -_HI, jnp.float32),
         jnp.full((_R, _L), -1, jnp.int32),
         jnp.zeros((_R, 1), jnp.int32),
         jnp.zeros((_R, _K), jnp.int32)))
    idx_ref[...] = final[3]


def _knn_topk(xt_pad, n_valid):
    n_pad, c = xt_pad.shape
    return pl.pallas_call(
        functools.partial(_topk_body, n_valid),
        grid=(n_pad // _R,),
        in_specs=[
            pl.BlockSpec((_R, c), lambda i: (i, 0)),
            pl.BlockSpec((n_pad, c), lambda i: (0, 0)),
        ],
        out_specs=pl.BlockSpec((_R, _K), lambda i: (i, 0)),
        out_shape=jax.ShapeDtypeStruct((n_pad, _K), jnp.int32),
        compiler_params=pltpu.CompilerParams(
            dimension_semantics=("arbitrary",)),
    )(xt_pad, xt_pad)


def _gather_max(xt, idx4, n_pad):
    c = xt.shape[1]
    mesh = plsc.VectorSubcoreMesh(core_axis_name="c", subcore_axis_name="s")

    @functools.partial(
        pl.kernel,
        mesh=mesh,
        out_type=jax.ShapeDtypeStruct((n_pad, c), jnp.float32),
        scratch_types=[
            pltpu.VMEM((_K, _CH, _CW), jnp.int32),
            pltpu.VMEM((_PW, c), jnp.float32),
            pltpu.VMEM((_PW, c), jnp.float32),
            pltpu.SemaphoreType.DMA,
        ],
    )
    def run(xt_hbm, idx_hbm, out_hbm, idx_v, acc_v, row_v, sem):
        wid = lax.axis_index("s") * 2 + lax.axis_index("c")
        base = wid * _PW
        pltpu.sync_copy(idx_hbm.at[:, wid], idx_v)
        for k in range(_K):
            dst = acc_v if k == 0 else row_v
            cps = [
                pltpu.async_copy(
                    xt_hbm.at[idx_v.at[k, j]],
                    dst.at[pl.ds(j * _CW, _CW)], sem)
                for j in range(_CH)
            ]
            for cp in cps:
                cp.wait()
            if k:
                @pl.loop(0, _PW)
                def _(r):
                    for c0 in range(0, c, 16):
                        sl = pl.ds(c0, 16)
                        acc_v[r, sl] = jnp.maximum(acc_v[r, sl],
                                                   row_v[r, sl])
        pltpu.sync_copy(acc_v, out_hbm.at[pl.ds(base, _PW)])

    return run(xt, idx4)


def _conv_body(x_ref, xm_ref, w1_ref, w2_ref, b_ref, o_ref):
    t1 = lax.dot_general(
        w1_ref[...], x_ref[...], (((1,), (0,)), ((), ())),
        precision=lax.Precision.HIGHEST,
        preferred_element_type=jnp.float32)
    t2 = lax.dot_general(
        w2_ref[...], xm_ref[...], (((1,), (1,)), ((), ())),
        precision=lax.Precision.HIGHEST,
        preferred_element_type=jnp.float32)
    o_ref[...] = jnp.maximum(t1 + t2 + b_ref[...], 0.0)


def _conv(x_pad, xmax, w1m, w2, b2d):
    c, n_pad = x_pad.shape
    out_c = w1m.shape[0]
    return pl.pallas_call(
        _conv_body,
        grid=(n_pad // _CONV_T,),
        in_specs=[
            pl.BlockSpec((c, _CONV_T), lambda i: (0, i)),
            pl.BlockSpec((_CONV_T, c), lambda i: (i, 0)),
            pl.BlockSpec((out_c, c), lambda i: (0, 0)),
            pl.BlockSpec((out_c, c), lambda i: (0, 0)),
            pl.BlockSpec((out_c, 1), lambda i: (0, 0)),
        ],
        out_specs=pl.BlockSpec((out_c, _CONV_T), lambda i: (0, i)),
        out_shape=jax.ShapeDtypeStruct((out_c, n_pad), jnp.float32),
        compiler_params=pltpu.CompilerParams(
            dimension_semantics=("arbitrary",)),
    )(x_pad, xmax, w1m, w2, b2d)


def kernel(x, W, b):
    _, c, n = x.shape
    align = _NW * _PW // math.gcd(_NW * _PW, _R) * _R
    n_pad = -(-n // align) * align

    xt = jnp.transpose(x[0])
    xt_pad = jnp.pad(xt, ((0, n_pad - n), (0, 0)))

    idx = _knn_topk(xt_pad, n)
    idx4 = jnp.transpose(idx).reshape(_K, _NW, n_pad // (_NW * _CW), _CW)

    xmax = _gather_max(xt, idx4, n_pad)

    x_pad = jnp.pad(x[0], ((0, 0), (0, n_pad - n)))
    w1m = W[:, :c] - W[:, c:]
    w2 = W[:, c:]
    out = _conv(x_pad, xmax, w1m, w2, b.reshape(-1, 1))
    return out[None, :, :n]


# speedup vs baseline: 1.1264x; 1.1264x over previous
import functools
import math

import jax
import jax.numpy as jnp
from jax import lax
from jax.experimental import pallas as pl
from jax.experimental.pallas import tpu as pltpu
from jax.experimental.pallas import tpu_sc as plsc

_K = 16
_R = 128
_NW = 32
_CW = 80
_CH = 4
_PW = _CW * _CH
_CONV_T = 512
_HI = float("inf")


def _topk_body(n_valid, xi_ref, xt_ref, idx_ref, dist_ref):
    xi = xi_ref[...]
    xi = xi / (jnp.sqrt(jnp.sum(xi * xi, axis=1, keepdims=True)) + 1e-12)
    sqi = jnp.sum(xi * xi, axis=1, keepdims=True)
    xj = xt_ref[...]
    xj = xj / (jnp.sqrt(jnp.sum(xj * xj, axis=1, keepdims=True)) + 1e-12)
    inner = lax.dot_general(
        xi, xj, (((1,), (1,)), ((), ())),
        precision=lax.Precision.DEFAULT,
        preferred_element_type=jnp.float32)
    ones = jnp.ones((1, xj.shape[1]), jnp.float32)
    sqj = lax.dot_general(
        ones, xj * xj, (((1,), (1,)), ((), ())),
        precision=lax.Precision.HIGHEST,
        preferred_element_type=jnp.float32)
    dist = sqi + (-2.0) * inner + sqj
    iota = lax.broadcasted_iota(jnp.int32, dist.shape, 1)
    dist_ref[...] = jnp.where(iota >= n_valid, _HI, dist)
    kiota = lax.broadcasted_iota(jnp.int32, (_R, _K), 1)

    def step(k, carry):
        prev, acc = carry
        d = jnp.where(iota == prev, _HI, dist_ref[...])
        dist_ref[...] = d
        m = jnp.min(d, axis=1, keepdims=True)
        idxk = jnp.min(jnp.where(d == m, iota, jnp.int32(2**30)),
                       axis=1, keepdims=True)
        return idxk, jnp.where(kiota == k, idxk, acc)

    idx_ref[...] = lax.fori_loop(
        0, _K, step,
        (jnp.full((_R, 1), -1, jnp.int32),
         jnp.zeros((_R, _K), jnp.int32)))[1]


def _knn_topk(xt_pad, n_valid):
    n_pad, c = xt_pad.shape
    return pl.pallas_call(
        functools.partial(_topk_body, n_valid),
        grid=(n_pad // _R,),
        in_specs=[
            pl.BlockSpec((_R, c), lambda i: (i, 0)),
            pl.BlockSpec((n_pad, c), lambda i: (0, 0)),
        ],
        out_specs=pl.BlockSpec((_R, _K), lambda i: (i, 0)),
        out_shape=jax.ShapeDtypeStruct((n_pad, _K), jnp.int32),
        scratch_shapes=[pltpu.VMEM((_R, n_pad), jnp.float32)],
        compiler_params=pltpu.CompilerParams(
            dimension_semantics=("parallel",)),
    )(xt_pad, xt_pad)


def _gather_max(xt, idx4, n_pad):
    c = xt.shape[1]
    mesh = plsc.VectorSubcoreMesh(core_axis_name="c", subcore_axis_name="s")

    @functools.partial(
        pl.kernel,
        mesh=mesh,
        out_type=jax.ShapeDtypeStruct((n_pad, c), jnp.float32),
        scratch_types=[
            pltpu.VMEM((_K, _CH, _CW), jnp.int32),
            pltpu.VMEM((_PW, c), jnp.float32),
            pltpu.VMEM((_PW, c), jnp.float32),
            pltpu.SemaphoreType.DMA,
        ],
    )
    def run(xt_hbm, idx_hbm, out_hbm, idx_v, acc_v, row_v, sem):
        wid = lax.axis_index("s") * 2 + lax.axis_index("c")
        base = wid * _PW
        pltpu.sync_copy(idx_hbm.at[:, wid], idx_v)
        for k in range(_K):
            dst = acc_v if k == 0 else row_v
            cps = [
                pltpu.async_copy(
                    xt_hbm.at[idx_v.at[k, j]],
                    dst.at[pl.ds(j * _CW, _CW)], sem)
                for j in range(_CH)
            ]
            for cp in cps:
                cp.wait()
            if k:
                @pl.loop(0, _PW)
                def _(r):
                    for c0 in range(0, c, 16):
                        sl = pl.ds(c0, 16)
                        acc_v[r, sl] = jnp.maximum(acc_v[r, sl],
                                                   row_v[r, sl])
        pltpu.sync_copy(acc_v, out_hbm.at[pl.ds(base, _PW)])

    return run(xt, idx4)


def _conv_body(x_ref, xm_ref, w1_ref, w2_ref, b_ref, o_ref):
    t1 = lax.dot_general(
        w1_ref[...], x_ref[...], (((1,), (0,)), ((), ())),
        precision=lax.Precision.HIGHEST,
        preferred_element_type=jnp.float32)
    t2 = lax.dot_general(
        w2_ref[...], xm_ref[...], (((1,), (1,)), ((), ())),
        precision=lax.Precision.HIGHEST,
        preferred_element_type=jnp.float32)
    o_ref[...] = jnp.maximum(t1 + t2 + b_ref[...], 0.0)


def _conv(x_pad, xmax, w1m, w2, b2d):
    c, n_pad = x_pad.shape
    out_c = w1m.shape[0]
    return pl.pallas_call(
        _conv_body,
        grid=(n_pad // _CONV_T,),
        in_specs=[
            pl.BlockSpec((c, _CONV_T), lambda i: (0, i)),
            pl.BlockSpec((_CONV_T, c), lambda i: (i, 0)),
            pl.BlockSpec((out_c, c), lambda i: (0, 0)),
            pl.BlockSpec((out_c, c), lambda i: (0, 0)),
            pl.BlockSpec((out_c, 1), lambda i: (0, 0)),
        ],
        out_specs=pl.BlockSpec((out_c, _CONV_T), lambda i: (0, i)),
        out_shape=jax.ShapeDtypeStruct((out_c, n_pad), jnp.float32),
        compiler_params=pltpu.CompilerParams(
            dimension_semantics=("arbitrary",)),
    )(x_pad, xmax, w1m, w2, b2d)


def kernel(x, W, b):
    _, c, n = x.shape
    align = _NW * _PW // math.gcd(_NW * _PW, _R) * _R
    n_pad = -(-n // align) * align

    xt = jnp.transpose(x[0])
    xt_pad = jnp.pad(xt, ((0, n_pad - n), (0, 0)))

    idx = _knn_topk(xt_pad, n)
    idx4 = jnp.transpose(idx).reshape(_K, _NW, n_pad // (_NW * _CW), _CW)

    xmax = _gather_max(xt, idx4, n_pad)

    x_pad = jnp.pad(x[0], ((0, 0), (0, n_pad - n)))
    w1m = W[:, :c] - W[:, c:]
    w2 = W[:, c:]
    out = _conv(x_pad, xmax, w1m, w2, b.reshape(-1, 1))
    return out[None, :, :n]


# speedup vs baseline: 1.3101x; 1.1631x over previous
import functools
import math

import jax
import jax.numpy as jnp
from jax import lax
from jax.experimental import pallas as pl
from jax.experimental.pallas import tpu as pltpu
from jax.experimental.pallas import tpu_sc as plsc

_K = 16
_R = 256
_NW = 32
_CW = 80
_CH = 4
_PW = _CW * _CH
_CONV_T = 512
_HI = float("inf")


def _topk_body(n_valid, xi_ref, xt_ref, idx_ref, dist_ref):
    xi = xi_ref[...]
    xi = xi / (jnp.sqrt(jnp.sum(xi * xi, axis=1, keepdims=True)) + 1e-12)
    sqi = jnp.sum(xi * xi, axis=1, keepdims=True)
    xj = xt_ref[...]
    xj = xj / (jnp.sqrt(jnp.sum(xj * xj, axis=1, keepdims=True)) + 1e-12)
    inner = lax.dot_general(
        xi, xj, (((1,), (1,)), ((), ())),
        precision=lax.Precision.DEFAULT,
        preferred_element_type=jnp.float32)
    ones = jnp.ones((1, xj.shape[1]), jnp.float32)
    sqj = lax.dot_general(
        ones, xj * xj, (((1,), (1,)), ((), ())),
        precision=lax.Precision.HIGHEST,
        preferred_element_type=jnp.float32)
    dist = sqi + (-2.0) * inner + sqj
    iota = lax.broadcasted_iota(jnp.int32, dist.shape, 1)
    dist_ref[...] = jnp.where(iota >= n_valid, _HI, dist)
    kiota = lax.broadcasted_iota(jnp.int32, (_R, _K), 1)

    def step(k, acc):
        d = dist_ref[...]
        m = jnp.min(d, axis=1, keepdims=True)
        idxk = jnp.min(jnp.where(d == m, iota, jnp.int32(2**30)),
                       axis=1, keepdims=True)
        dist_ref[...] = jnp.where(iota == idxk, _HI, d)
        return jnp.where(kiota == k, idxk, acc)

    idx_ref[...] = lax.fori_loop(0, _K, step,
                                 jnp.zeros((_R, _K), jnp.int32))


def _knn_topk(xt_pad, n_valid):
    n_pad, c = xt_pad.shape
    return pl.pallas_call(
        functools.partial(_topk_body, n_valid),
        grid=(n_pad // _R,),
        in_specs=[
            pl.BlockSpec((_R, c), lambda i: (i, 0)),
            pl.BlockSpec((n_pad, c), lambda i: (0, 0)),
        ],
        out_specs=pl.BlockSpec((_R, _K), lambda i: (i, 0)),
        out_shape=jax.ShapeDtypeStruct((n_pad, _K), jnp.int32),
        scratch_shapes=[pltpu.VMEM((_R, n_pad), jnp.float32)],
        compiler_params=pltpu.CompilerParams(
            dimension_semantics=("parallel",)),
    )(xt_pad, xt_pad)


def _gather_max(xt, idx4, n_pad):
    c = xt.shape[1]
    mesh = plsc.VectorSubcoreMesh(core_axis_name="c", subcore_axis_name="s")

    @functools.partial(
        pl.kernel,
        mesh=mesh,
        out_type=jax.ShapeDtypeStruct((n_pad, c), jnp.float32),
        scratch_types=[
            pltpu.VMEM((_K, _CH, _CW), jnp.int32),
            pltpu.VMEM((_PW, c), jnp.float32),
            pltpu.VMEM((_PW, c), jnp.float32),
            pltpu.SemaphoreType.DMA,
        ],
    )
    def run(xt_hbm, idx_hbm, out_hbm, idx_v, acc_v, row_v, sem):
        wid = lax.axis_index("s") * 2 + lax.axis_index("c")
        base = wid * _PW
        pltpu.sync_copy(idx_hbm.at[:, wid], idx_v)
        for k in range(_K):
            dst = acc_v if k == 0 else row_v
            cps = [
                pltpu.async_copy(
                    xt_hbm.at[idx_v.at[k, j]],
                    dst.at[pl.ds(j * _CW, _CW)], sem)
                for j in range(_CH)
            ]
            for cp in cps:
                cp.wait()
            if k:
                @pl.loop(0, _PW)
                def _(r):
                    for c0 in range(0, c, 16):
                        sl = pl.ds(c0, 16)
                        acc_v[r, sl] = jnp.maximum(acc_v[r, sl],
                                                   row_v[r, sl])
        pltpu.sync_copy(acc_v, out_hbm.at[pl.ds(base, _PW)])

    return run(xt, idx4)


def _conv_body(x_ref, xm_ref, w1_ref, w2_ref, b_ref, o_ref):
    t1 = lax.dot_general(
        w1_ref[...], x_ref[...], (((1,), (0,)), ((), ())),
        precision=lax.Precision.HIGHEST,
        preferred_element_type=jnp.float32)
    t2 = lax.dot_general(
        w2_ref[...], xm_ref[...], (((1,), (1,)), ((), ())),
        precision=lax.Precision.HIGHEST,
        preferred_element_type=jnp.float32)
    o_ref[...] = jnp.maximum(t1 + t2 + b_ref[...], 0.0)


def _conv(x_pad, xmax, w1m, w2, b2d):
    c, n_pad = x_pad.shape
    out_c = w1m.shape[0]
    return pl.pallas_call(
        _conv_body,
        grid=(n_pad // _CONV_T,),
        in_specs=[
            pl.BlockSpec((c, _CONV_T), lambda i: (0, i)),
            pl.BlockSpec((_CONV_T, c), lambda i: (i, 0)),
            pl.BlockSpec((out_c, c), lambda i: (0, 0)),
            pl.BlockSpec((out_c, c), lambda i: (0, 0)),
            pl.BlockSpec((out_c, 1), lambda i: (0, 0)),
        ],
        out_specs=pl.BlockSpec((out_c, _CONV_T), lambda i: (0, i)),
        out_shape=jax.ShapeDtypeStruct((out_c, n_pad), jnp.float32),
        compiler_params=pltpu.CompilerParams(
            dimension_semantics=("arbitrary",)),
    )(x_pad, xmax, w1m, w2, b2d)


def kernel(x, W, b):
    _, c, n = x.shape
    align = _NW * _PW // math.gcd(_NW * _PW, _R) * _R
    n_pad = -(-n // align) * align

    xt = jnp.transpose(x[0])
    xt_pad = jnp.pad(xt, ((0, n_pad - n), (0, 0)))

    idx = _knn_topk(xt_pad, n)
    idx4 = jnp.transpose(idx).reshape(_K, _NW, n_pad // (_NW * _CW), _CW)

    xmax = _gather_max(xt, idx4, n_pad)

    x_pad = jnp.pad(x[0], ((0, 0), (0, n_pad - n)))
    w1m = W[:, :c] - W[:, c:]
    w2 = W[:, c:]
    out = _conv(x_pad, xmax, w1m, w2, b.reshape(-1, 1))
    return out[None, :, :n]


# speedup vs baseline: 1.4442x; 1.1023x over previous
import functools
import math

import jax
import jax.numpy as jnp
from jax import lax
from jax.experimental import pallas as pl
from jax.experimental.pallas import tpu as pltpu
from jax.experimental.pallas import tpu_sc as plsc

_K = 16
_R = 512
_NW = 32
_CW = 80
_CH = 4
_PW = _CW * _CH
_CONV_T = 512
_HI = float("inf")


def _topk_body(n_valid, xi_ref, xt_ref, idx_ref, dist_ref):
    xi = xi_ref[...]
    xi = xi / (jnp.sqrt(jnp.sum(xi * xi, axis=1, keepdims=True)) + 1e-12)
    sqi = jnp.sum(xi * xi, axis=1, keepdims=True)
    xj = xt_ref[...]
    xj = xj / (jnp.sqrt(jnp.sum(xj * xj, axis=1, keepdims=True)) + 1e-12)
    inner = lax.dot_general(
        xi, xj, (((1,), (1,)), ((), ())),
        precision=lax.Precision.DEFAULT,
        preferred_element_type=jnp.float32)
    ones = jnp.ones((1, xj.shape[1]), jnp.float32)
    sqj = lax.dot_general(
        ones, xj * xj, (((1,), (1,)), ((), ())),
        precision=lax.Precision.HIGHEST,
        preferred_element_type=jnp.float32)
    dist = sqi + (-2.0) * inner + sqj
    iota = lax.broadcasted_iota(jnp.int32, dist.shape, 1)
    dist_ref[...] = jnp.where(iota >= n_valid, _HI, dist)
    kiota = lax.broadcasted_iota(jnp.int32, (_R, _K), 1)

    def step(k, acc):
        d = dist_ref[...]
        m = jnp.min(d, axis=1, keepdims=True)
        idxk = jnp.min(jnp.where(d == m, iota, jnp.int32(2**30)),
                       axis=1, keepdims=True)
        dist_ref[...] = jnp.where(iota == idxk, _HI, d)
        return jnp.where(kiota == k, idxk, acc)

    idx_ref[...] = lax.fori_loop(0, _K, step,
                                 jnp.zeros((_R, _K), jnp.int32))


def _knn_topk(xt_pad, n_valid):
    n_pad, c = xt_pad.shape
    return pl.pallas_call(
        functools.partial(_topk_body, n_valid),
        grid=(n_pad // _R,),
        in_specs=[
            pl.BlockSpec((_R, c), lambda i: (i, 0)),
            pl.BlockSpec((n_pad, c), lambda i: (0, 0)),
        ],
        out_specs=pl.BlockSpec((_R, _K), lambda i: (i, 0)),
        out_shape=jax.ShapeDtypeStruct((n_pad, _K), jnp.int32),
        scratch_shapes=[pltpu.VMEM((_R, n_pad), jnp.float32)],
        compiler_params=pltpu.CompilerParams(
            dimension_semantics=("parallel",)),
    )(xt_pad, xt_pad)


def _gather_max(xt, idx4, n_pad):
    c = xt.shape[1]
    mesh = plsc.VectorSubcoreMesh(core_axis_name="c", subcore_axis_name="s")

    @functools.partial(
        pl.kernel,
        mesh=mesh,
        out_type=jax.ShapeDtypeStruct((n_pad, c), jnp.float32),
        scratch_types=[
            pltpu.VMEM((_K, _CH, _CW), jnp.int32),
            pltpu.VMEM((_PW, c), jnp.float32),
            pltpu.VMEM((_PW, c), jnp.float32),
            pltpu.SemaphoreType.DMA,
        ],
    )
    def run(xt_hbm, idx_hbm, out_hbm, idx_v, acc_v, row_v, sem):
        wid = lax.axis_index("s") * 2 + lax.axis_index("c")
        base = wid * _PW
        pltpu.sync_copy(idx_hbm.at[:, wid], idx_v)
        for k in range(_K):
            dst = acc_v if k == 0 else row_v
            cps = [
                pltpu.async_copy(
                    xt_hbm.at[idx_v.at[k, j]],
                    dst.at[pl.ds(j * _CW, _CW)], sem)
                for j in range(_CH)
            ]
            for cp in cps:
                cp.wait()
            if k:
                @pl.loop(0, _PW)
                def _(r):
                    for c0 in range(0, c, 16):
                        sl = pl.ds(c0, 16)
                        acc_v[r, sl] = jnp.maximum(acc_v[r, sl],
                                                   row_v[r, sl])
        pltpu.sync_copy(acc_v, out_hbm.at[pl.ds(base, _PW)])

    return run(xt, idx4)


def _conv_body(x_ref, xm_ref, w1_ref, w2_ref, b_ref, o_ref):
    t1 = lax.dot_general(
        w1_ref[...], x_ref[...], (((1,), (0,)), ((), ())),
        precision=lax.Precision.HIGHEST,
        preferred_element_type=jnp.float32)
    t2 = lax.dot_general(
        w2_ref[...], xm_ref[...], (((1,), (1,)), ((), ())),
        precision=lax.Precision.HIGHEST,
        preferred_element_type=jnp.float32)
    o_ref[...] = jnp.maximum(t1 + t2 + b_ref[...], 0.0)


def _conv(x_pad, xmax, w1m, w2, b2d):
    c, n_pad = x_pad.shape
    out_c = w1m.shape[0]
    return pl.pallas_call(
        _conv_body,
        grid=(n_pad // _CONV_T,),
        in_specs=[
            pl.BlockSpec((c, _CONV_T), lambda i: (0, i)),
            pl.BlockSpec((_CONV_T, c), lambda i: (i, 0)),
            pl.BlockSpec((out_c, c), lambda i: (0, 0)),
            pl.BlockSpec((out_c, c), lambda i: (0, 0)),
            pl.BlockSpec((out_c, 1), lambda i: (0, 0)),
        ],
        out_specs=pl.BlockSpec((out_c, _CONV_T), lambda i: (0, i)),
        out_shape=jax.ShapeDtypeStruct((out_c, n_pad), jnp.float32),
        compiler_params=pltpu.CompilerParams(
            dimension_semantics=("arbitrary",)),
    )(x_pad, xmax, w1m, w2, b2d)


def kernel(x, W, b):
    _, c, n = x.shape
    align = _NW * _PW // math.gcd(_NW * _PW, _R) * _R
    n_pad = -(-n // align) * align

    xt = jnp.transpose(x[0])
    xt_pad = jnp.pad(xt, ((0, n_pad - n), (0, 0)))

    idx = _knn_topk(xt_pad, n)
    idx4 = jnp.transpose(idx).reshape(_K, _NW, n_pad // (_NW * _CW), _CW)

    xmax = _gather_max(xt, idx4, n_pad)

    x_pad = jnp.pad(x[0], ((0, 0), (0, n_pad - n)))
    w1m = W[:, :c] - W[:, c:]
    w2 = W[:, c:]
    out = _conv(x_pad, xmax, w1m, w2, b.reshape(-1, 1))
    return out[None, :, :n]


# speedup vs baseline: 1.4652x; 1.0145x over previous
import functools
import math

import jax
import jax.numpy as jnp
from jax import lax
from jax.experimental import pallas as pl
from jax.experimental.pallas import tpu as pltpu
from jax.experimental.pallas import tpu_sc as plsc

_K = 16
_R = 512
_NW = 32
_CW = 80
_CH = 4
_PW = _CW * _CH
_CONV_T = 512
_HI = float("inf")


def _topk_body(n_valid, xi_ref, xt_ref, idx_ref, dist_ref):
    xi = xi_ref[...]
    xi = xi / (jnp.sqrt(jnp.sum(xi * xi, axis=1, keepdims=True)) + 1e-12)
    sqi = jnp.sum(xi * xi, axis=1, keepdims=True)
    xj = xt_ref[...]
    xj = xj / (jnp.sqrt(jnp.sum(xj * xj, axis=1, keepdims=True)) + 1e-12)
    inner = lax.dot_general(
        xi, xj, (((1,), (1,)), ((), ())),
        precision=lax.Precision.DEFAULT,
        preferred_element_type=jnp.float32)
    ones = jnp.ones((1, xj.shape[1]), jnp.float32)
    sqj = lax.dot_general(
        ones, xj * xj, (((1,), (1,)), ((), ())),
        precision=lax.Precision.HIGHEST,
        preferred_element_type=jnp.float32)
    dist = sqi + (-2.0) * inner + sqj
    iota = lax.broadcasted_iota(jnp.int32, dist.shape, 1)
    dist_ref[...] = jnp.where(iota >= n_valid, _HI, dist)
    kiota = lax.broadcasted_iota(jnp.int32, (_R, _K), 1)

    def step(k, acc):
        d = dist_ref[...]
        m = jnp.min(d, axis=1, keepdims=True)
        idxk = jnp.min(jnp.where(d == m, iota, jnp.int32(2**30)),
                       axis=1, keepdims=True)
        dist_ref[...] = jnp.where(iota == idxk, _HI, d)
        return jnp.where(kiota == k, idxk, acc)

    idx_ref[...] = lax.fori_loop(0, _K, step,
                                 jnp.zeros((_R, _K), jnp.int32))


def _knn_topk(xt_pad, n_valid):
    n_pad, c = xt_pad.shape
    return pl.pallas_call(
        functools.partial(_topk_body, n_valid),
        grid=(n_pad // _R,),
        in_specs=[
            pl.BlockSpec((_R, c), lambda i: (i, 0)),
            pl.BlockSpec((n_pad, c), lambda i: (0, 0)),
        ],
        out_specs=pl.BlockSpec((_R, _K), lambda i: (i, 0)),
        out_shape=jax.ShapeDtypeStruct((n_pad, _K), jnp.int32),
        scratch_shapes=[pltpu.VMEM((_R, n_pad), jnp.float32)],
        compiler_params=pltpu.CompilerParams(
            dimension_semantics=("parallel",)),
    )(xt_pad, xt_pad)


def _gather_max(xt, idx4, n_pad):
    c = xt.shape[1]
    mesh = plsc.VectorSubcoreMesh(core_axis_name="c", subcore_axis_name="s")

    @functools.partial(
        pl.kernel,
        mesh=mesh,
        out_type=jax.ShapeDtypeStruct((n_pad, c), jnp.float32),
        scratch_types=[
            pltpu.VMEM((_K, _CH, _CW), jnp.int32),
            pltpu.VMEM((_PW, c), jnp.float32),
            pltpu.VMEM((_CW, c), jnp.float32),
            pltpu.VMEM((_CW, c), jnp.float32),
            pltpu.SemaphoreType.DMA,
            pltpu.SemaphoreType.DMA,
        ],
    )
    def run(xt_hbm, idx_hbm, out_hbm, idx_v, acc_v, rowa_v, rowb_v,
            sema, semb):
        wid = lax.axis_index("s") * 2 + lax.axis_index("c")
        base = wid * _PW
        pltpu.sync_copy(idx_hbm.at[:, wid], idx_v)
        cps = [
            pltpu.async_copy(
                xt_hbm.at[idx_v.at[0, j]],
                acc_v.at[pl.ds(j * _CW, _CW)], sema)
            for j in range(_CH)
        ]
        for cp in cps:
            cp.wait()
        items = [(k, j) for k in range(1, _K) for j in range(_CH)]
        bufs, sems = (rowa_v, rowb_v), (sema, semb)
        prev = None
        for t, (k, j) in enumerate(items):
            cp = pltpu.async_copy(
                xt_hbm.at[idx_v.at[k, j]], bufs[t % 2], sems[t % 2])
            if prev is not None:
                pj, pcp = prev
                pcp.wait()
                pbuf = bufs[(t - 1) % 2]
                pbase = pj * _CW

                @pl.loop(0, _CW)
                def _(r):
                    for c0 in range(0, c, 16):
                        sl = pl.ds(c0, 16)
                        acc_v[pbase + r, sl] = jnp.maximum(
                            acc_v[pbase + r, sl], pbuf[r, sl])
            prev = (j, cp)
        pj, pcp = prev
        pcp.wait()
        pbuf = bufs[(len(items) - 1) % 2]
        pbase = pj * _CW

        @pl.loop(0, _CW)
        def _(r):
            for c0 in range(0, c, 16):
                sl = pl.ds(c0, 16)
                acc_v[pbase + r, sl] = jnp.maximum(acc_v[pbase + r, sl],
                                                   pbuf[r, sl])
        pltpu.sync_copy(acc_v, out_hbm.at[pl.ds(base, _PW)])

    return run(xt, idx4)


def _conv_body(x_ref, xm_ref, w1_ref, w2_ref, b_ref, o_ref):
    t1 = lax.dot_general(
        w1_ref[...], x_ref[...], (((1,), (0,)), ((), ())),
        precision=lax.Precision.HIGHEST,
        preferred_element_type=jnp.float32)
    t2 = lax.dot_general(
        w2_ref[...], xm_ref[...], (((1,), (1,)), ((), ())),
        precision=lax.Precision.HIGHEST,
        preferred_element_type=jnp.float32)
    o_ref[...] = jnp.maximum(t1 + t2 + b_ref[...], 0.0)


def _conv(x_pad, xmax, w1m, w2, b2d):
    c, n_pad = x_pad.shape
    out_c = w1m.shape[0]
    return pl.pallas_call(
        _conv_body,
        grid=(n_pad // _CONV_T,),
        in_specs=[
            pl.BlockSpec((c, _CONV_T), lambda i: (0, i)),
            pl.BlockSpec((_CONV_T, c), lambda i: (i, 0)),
            pl.BlockSpec((out_c, c), lambda i: (0, 0)),
            pl.BlockSpec((out_c, c), lambda i: (0, 0)),
            pl.BlockSpec((out_c, 1), lambda i: (0, 0)),
        ],
        out_specs=pl.BlockSpec((out_c, _CONV_T), lambda i: (0, i)),
        out_shape=jax.ShapeDtypeStruct((out_c, n_pad), jnp.float32),
        compiler_params=pltpu.CompilerParams(
            dimension_semantics=("arbitrary",)),
    )(x_pad, xmax, w1m, w2, b2d)


def kernel(x, W, b):
    _, c, n = x.shape
    align = _NW * _PW // math.gcd(_NW * _PW, _R) * _R
    n_pad = -(-n // align) * align

    xt = jnp.transpose(x[0])
    xt_pad = jnp.pad(xt, ((0, n_pad - n), (0, 0)))

    idx = _knn_topk(xt_pad, n)
    idx4 = jnp.transpose(idx).reshape(_K, _NW, n_pad // (_NW * _CW), _CW)

    xmax = _gather_max(xt, idx4, n_pad)

    x_pad = jnp.pad(x[0], ((0, 0), (0, n_pad - n)))
    w1m = W[:, :c] - W[:, c:]
    w2 = W[:, c:]
    out = _conv(x_pad, xmax, w1m, w2, b.reshape(-1, 1))
    return out[None, :, :n]
